# Initial kernel scaffold; baseline (speedup 1.0000x reference)
import jax
import jax.numpy as jnp
from jax.experimental import pallas as pl


def kernel(x, y, t, w):
    raise NotImplementedError("write your pallas kernel here")


# speedup vs baseline: 2.6479x; 2.6479x over previous
import functools

import jax
import jax.numpy as jnp
from jax import lax
from jax.experimental import pallas as pl
from jax.experimental.pallas import tpu as pltpu

B = 2048
IN = 256
OUT = 128
LANE = 128
NCHUNK = B // LANE


def _xor_perm(a, j):
    iota = lax.broadcasted_iota(jnp.int32, a.shape, 0)
    bit = (iota & j) != 0
    up = jnp.roll(a, j, axis=0)
    dn = jnp.roll(a, -j, axis=0)
    return jnp.where(bit, up, dn)


def _loss_body(xT_ref, tT_ref, w_ref, out_ref, s_ref):
    f32 = jnp.float32

    dacc = xT_ref[:, 0:LANE]
    for c in range(1, NCHUNK):
        dacc = dacc + xT_ref[:, c * LANE:(c + 1) * LANE]
    denom = jnp.sum(dacc, axis=1, keepdims=True)

    s_ref[...] = jnp.zeros((IN, OUT), f32)
    onehot_iota = lax.broadcasted_iota(jnp.int32, (1, OUT), 1)

    def obody(o, carry):
        trow = tT_ref[pl.ds(o, 1), :]
        acc = jnp.minimum(xT_ref[:, 0:LANE], trow[:, 0:LANE])
        for c in range(1, NCHUNK):
            acc = acc + jnp.minimum(
                xT_ref[:, c * LANE:(c + 1) * LANE],
                trow[:, c * LANE:(c + 1) * LANE])
        scol = jnp.sum(acc, axis=1, keepdims=True)
        onehot = (onehot_iota == o).astype(f32)
        s_ref[...] += scol * onehot
        return carry

    lax.fori_loop(0, OUT, obody, 0)

    s = s_ref[...]
    score = jnp.where(denom == 0.0, jnp.float32(1.0), s / denom)

    iota0 = lax.broadcasted_iota(jnp.int32, (IN, OUT), 0)
    key = score
    idx = iota0
    pay = w_ref[...]
    for k in [2, 4, 8, 16, 32, 64, 128, 256]:
        j = k // 2
        while j >= 1:
            kp = _xor_perm(key, j)
            ip = _xor_perm(idx, j)
            pp = _xor_perm(pay, j)
            is_lower = (iota0 & j) == 0
            d = (iota0 & k) == 0
            before = (key > kp) | ((key == kp) & (idx < ip))
            keep = (before == d) == is_lower
            key = jnp.where(keep, key, kp)
            idx = jnp.where(keep, idx, ip)
            pay = jnp.where(keep, pay, pp)
            j //= 2
    target_w = pay

    sw = w_ref[...]
    for k in [2, 4, 8, 16, 32, 64, 128, 256]:
        j = k // 2
        while j >= 1:
            swp = _xor_perm(sw, j)
            is_lower = (iota0 & j) == 0
            d = (iota0 & k) == 0
            hi = jnp.maximum(sw, swp)
            lo = jnp.minimum(sw, swp)
            sw = jnp.where(is_lower == d, hi, lo)
            j //= 2
    sorted_w = sw

    diff = sorted_w - target_w
    sq = diff * diff
    total = jnp.sum(jnp.sum(sq, axis=0, keepdims=True), axis=1, keepdims=True)
    out_ref[...] = total / jnp.float32(IN * OUT)


@functools.partial(jax.jit, static_argnames=("interpret",))
def _run(x, t, w, interpret=False):
    xT = x.T
    tT = t.T
    out = pl.pallas_call(
        _loss_body,
        out_shape=jax.ShapeDtypeStruct((1, 1), jnp.float32),
        scratch_shapes=[pltpu.VMEM((IN, OUT), jnp.float32)],
        interpret=interpret,
    )(xT, tT, w)
    return out[0, 0]


def kernel(x, y, t, w):
    del y
    return _run(x, t, w)


# speedup vs baseline: 3.0636x; 1.1570x over previous
import functools

import jax
import jax.numpy as jnp
from jax import lax
from jax.experimental import pallas as pl
from jax.experimental.pallas import tpu as pltpu

B = 2048
IN = 256
OUT = 128
LANE = 128
NCHUNK = B // LANE


def _xor_perm(a, j):
    iota = lax.broadcasted_iota(jnp.int32, a.shape, 0)
    bit = (iota & j) != 0
    up = jnp.roll(a, j, axis=0)
    dn = jnp.roll(a, -j, axis=0)
    return jnp.where(bit, up, dn)


def _loss_body(xT_ref, tT_ref, w_ref, out_ref, s_ref):
    f32 = jnp.float32

    dacc = xT_ref[:, 0:LANE]
    for c in range(1, NCHUNK):
        dacc = dacc + xT_ref[:, c * LANE:(c + 1) * LANE]
    denom = jnp.sum(dacc, axis=1, keepdims=True)

    s_ref[...] = jnp.zeros((IN, OUT), f32)
    onehot_iota = lax.broadcasted_iota(jnp.int32, (1, OUT), 1)

    def obody(i, carry):
        o0 = i * 2
        o1 = o0 + 1
        t0 = tT_ref[pl.ds(o0, 1), :]
        t1 = tT_ref[pl.ds(o1, 1), :]
        xc = xT_ref[:, 0:LANE]
        acc0 = jnp.minimum(xc, t0[:, 0:LANE])
        acc1 = jnp.minimum(xc, t1[:, 0:LANE])
        for c in range(1, NCHUNK):
            xc = xT_ref[:, c * LANE:(c + 1) * LANE]
            acc0 = acc0 + jnp.minimum(xc, t0[:, c * LANE:(c + 1) * LANE])
            acc1 = acc1 + jnp.minimum(xc, t1[:, c * LANE:(c + 1) * LANE])
        s0 = jnp.sum(acc0, axis=1, keepdims=True)
        s1 = jnp.sum(acc1, axis=1, keepdims=True)
        oh0 = (onehot_iota == o0).astype(f32)
        oh1 = (onehot_iota == o1).astype(f32)
        s_ref[...] += s0 * oh0 + s1 * oh1
        return carry

    lax.fori_loop(0, OUT // 2, obody, 0)

    s = s_ref[...]
    score = jnp.where(denom == 0.0, jnp.float32(1.0), s / denom)

    iota0 = lax.broadcasted_iota(jnp.int32, (IN, OUT), 0)
    key = score
    idx = iota0
    pay = w_ref[...]
    for k in [2, 4, 8, 16, 32, 64, 128, 256]:
        j = k // 2
        while j >= 1:
            kp = _xor_perm(key, j)
            ip = _xor_perm(idx, j)
            pp = _xor_perm(pay, j)
            is_lower = (iota0 & j) == 0
            d = (iota0 & k) == 0
            before = (key > kp) | ((key == kp) & (idx < ip))
            keep = (before == d) == is_lower
            key = jnp.where(keep, key, kp)
            idx = jnp.where(keep, idx, ip)
            pay = jnp.where(keep, pay, pp)
            j //= 2
    target_w = pay

    sw = w_ref[...]
    for k in [2, 4, 8, 16, 32, 64, 128, 256]:
        j = k // 2
        while j >= 1:
            swp = _xor_perm(sw, j)
            is_lower = (iota0 & j) == 0
            d = (iota0 & k) == 0
            hi = jnp.maximum(sw, swp)
            lo = jnp.minimum(sw, swp)
            sw = jnp.where(is_lower == d, hi, lo)
            j //= 2
    sorted_w = sw

    diff = sorted_w - target_w
    sq = diff * diff
    total = jnp.sum(jnp.sum(sq, axis=0, keepdims=True), axis=1, keepdims=True)
    out_ref[...] = total / jnp.float32(IN * OUT)


@functools.partial(jax.jit, static_argnames=("interpret",))
def _run(x, t, w, interpret=False):
    xT = x.T
    tT = t.T
    out = pl.pallas_call(
        _loss_body,
        out_shape=jax.ShapeDtypeStruct((1, 1), jnp.float32),
        scratch_shapes=[pltpu.VMEM((IN, OUT), jnp.float32)],
        interpret=interpret,
    )(xT, tT, w)
    return out[0, 0]


def kernel(x, y, t, w):
    del y
    return _run(x, t, w)


# speedup vs baseline: 3.0656x; 1.0007x over previous
import functools

import jax
import jax.numpy as jnp
from jax import lax
from jax.experimental import pallas as pl
from jax.experimental.pallas import tpu as pltpu

B = 2048
IN = 256
OUT = 128
LANE = 128
NCHUNK = B // LANE
OGRP = 8


def _xor_perm1(a, j):
    iota = lax.broadcasted_iota(jnp.int32, a.shape, 1)
    bit = (iota & j) != 0
    up = jnp.roll(a, j, axis=1)
    dn = jnp.roll(a, -j, axis=1)
    return jnp.where(bit, up, dn)


def _loss_body(xT_ref, tT_ref, wT_ref, out_ref, sT_ref):
    f32 = jnp.float32
    ones_col = jnp.ones((LANE, 1), f32)

    dacc = xT_ref[:, 0:LANE]
    for c in range(1, NCHUNK):
        dacc = dacc + xT_ref[:, c * LANE:(c + 1) * LANE]
    denomT = lax.dot_general(ones_col, dacc, (((0,), (1,)), ((), ())),
                             preferred_element_type=f32)

    def gbody(g, carry):
        o0 = g * OGRP
        for ih in range(2):
            rs = slice(ih * (IN // 2), (ih + 1) * (IN // 2))
            accs = [None] * OGRP
            for c in range(NCHUNK):
                cs = slice(c * LANE, (c + 1) * LANE)
                xc = xT_ref[rs, cs]
                t8 = tT_ref[pl.ds(o0, OGRP), cs]
                for r in range(OGRP):
                    trow = lax.slice(t8, (r, 0), (r + 1, LANE))
                    m = jnp.minimum(xc, trow)
                    accs[r] = m if c == 0 else accs[r] + m
            srows = [lax.dot_general(ones_col, accs[r], (((0,), (1,)), ((), ())),
                                     preferred_element_type=f32)
                     for r in range(OGRP)]
            sblkT = jnp.concatenate(srows, axis=0)
            sT_ref[pl.ds(o0, OGRP), rs] = sblkT
        return carry

    lax.fori_loop(0, OUT // OGRP, gbody, 0)

    sT = sT_ref[...]
    scoreT = jnp.where(denomT == 0.0, jnp.float32(1.0), sT / denomT)

    iota1 = lax.broadcasted_iota(jnp.int32, (OUT, IN), 1)
    key = scoreT
    pay = wT_ref[...]
    for k in [2, 4, 8, 16, 32, 64, 128, 256]:
        j = k // 2
        while j >= 1:
            kp = _xor_perm1(key, j)
            pp = _xor_perm1(pay, j)
            is_lower = (iota1 & j) == 0
            before = (key > kp) | (is_lower & (key == kp))
            pbits = iota1 & (k + j)
            flip = (pbits == k) | (pbits == j)
            keep = before != flip
            key = jnp.where(keep, key, kp)
            pay = jnp.where(keep, pay, pp)
            j //= 2
    target_w = pay

    sw = wT_ref[...]
    for k in [2, 4, 8, 16, 32, 64, 128, 256]:
        j = k // 2
        while j >= 1:
            swp = _xor_perm1(sw, j)
            is_lower = (iota1 & j) == 0
            d = (iota1 & k) == 0
            hi = jnp.maximum(sw, swp)
            lo = jnp.minimum(sw, swp)
            sw = jnp.where(is_lower == d, hi, lo)
            j //= 2
    sorted_w = sw

    diff = sorted_w - target_w
    sq = diff * diff
    total = jnp.sum(jnp.sum(sq, axis=0, keepdims=True), axis=1, keepdims=True)
    out_ref[...] = total / jnp.float32(IN * OUT)


@functools.partial(jax.jit, static_argnames=("interpret",))
def _run(x, t, w, interpret=False):
    xT = x.T
    tT = t.T
    wT = w.T
    out = pl.pallas_call(
        _loss_body,
        out_shape=jax.ShapeDtypeStruct((1, 1), jnp.float32),
        scratch_shapes=[pltpu.VMEM((OUT, IN), jnp.float32)],
        interpret=interpret,
    )(xT, tT, wT)
    return out[0, 0]


def kernel(x, y, t, w):
    del y
    return _run(x, t, w)
